# baseline (speedup 1.0000x reference)
import functools

import jax
import jax.numpy as jnp
from jax.experimental import pallas as pl
from jax.experimental.pallas import tpu as pltpu

_BN_SCALE = 1.0 / (1.0 + 1e-5) ** 0.5


def _head_body(emb_ref, W1_ref, b1_ref, W2_ref, b2_ref, W3_ref, b3_ref,
               Wo_ref, bo_ref, s1_ref, t1_ref, s2_ref, t2_ref, s3_ref, t3_ref,
               out_ref):
    o = jnp.maximum(emb_ref[...] @ W1_ref[...] + b1_ref[...], 0.0)
    o = o * s1_ref[...] + t1_ref[...]
    o = jnp.maximum(o @ W2_ref[...] + b2_ref[...], 0.0)
    o = o * s2_ref[...] + t2_ref[...]
    o = jnp.maximum(o @ W3_ref[...] + b3_ref[...], 0.0)
    o = o * s3_ref[...] + t3_ref[...]
    out_ref[...] = o @ Wo_ref[...] + bo_ref[...]


def _mlp_head(emb, W1, b1, W2, b2, W3, b3, Wo, bo, g1, be1, g2, be2, g3, be3):
    G = emb.shape[0]
    Gp = 1024
    embp = jnp.zeros((Gp, 16), jnp.float32).at[:G].set(emb)
    out = pl.pallas_call(
        _head_body,
        out_shape=jax.ShapeDtypeStruct((Gp, 1), jnp.float32),
    )(embp, W1, b1[None, :], W2, b2[None, :], W3, b3[None, :], Wo, bo[None, :],
      (g1 * _BN_SCALE)[None, :], be1[None, :],
      (g2 * _BN_SCALE)[None, :], be2[None, :],
      (g3 * _BN_SCALE)[None, :], be3[None, :])
    return out[:G, 0]


def _conv(x, ei, ea, Wn, bn, Wr, br):
    m = jnp.tanh(jnp.concatenate([x[ei[1]], ea], axis=1) @ Wn + bn)
    agg = jax.ops.segment_sum(m, ei[0], num_segments=x.shape[0])
    return jnp.tanh(x @ Wr + br) + agg


def kernel(x, edge_index, edge_attr, batch, W_neg1, b_neg1, W_root1, b_root1,
           W_neg2, b_neg2, W_root2, b_root2, Wg1, bg1, Wg2, bg2, Wg3, bg3,
           W1, b1, W2, b2, W3, b3, Wo, bo, g1, be1, g2, be2, g3, be3):
    G = 1000
    h = _conv(x, edge_index, edge_attr, W_neg1, b_neg1, W_root1, b_root1)
    h = _conv(h, edge_index, edge_attr, W_neg2, b_neg2, W_root2, b_root2)
    gate = jax.nn.relu(h @ Wg1 + bg1)
    gate = jax.nn.relu(gate @ Wg2 + bg2)
    gate = gate @ Wg3 + bg3
    gmax = jax.ops.segment_max(gate, batch, num_segments=G)
    gexp = jnp.exp(gate - gmax[batch])
    gden = jax.ops.segment_sum(gexp, batch, num_segments=G)
    att = gexp / (gden[batch] + 1e-16)
    emb = jax.ops.segment_sum(att * h, batch, num_segments=G)
    o = _mlp_head(emb, W1, b1, W2, b2, W3, b3, Wo, bo, g1, be1, g2, be2, g3, be3)
    return (o, att)


# speedup vs baseline: 1.9728x; 1.9728x over previous
import functools

import jax
import jax.numpy as jnp
from jax import lax
from jax.experimental import pallas as pl
from jax.experimental.pallas import tpu as pltpu
from jax.experimental.pallas import tpu_sc as plsc

N = 50000
E = 1600000
G = 1000

NC = 2
NS = 16
CE = 256
K = CE // 128
NPAD = 50176
EP = NS * 392 * CE
ROWS_PER_TILE = NPAD // NS
DUMMY_DST = 50100

_BN_SCALE = 1.0 / (1.0 + 1e-5) ** 0.5


def _tanh_vreg(z):
    ez = jnp.exp(z + z)
    return 1.0 - 2.0 / (ez + 1.0)


def _make_sc_conv(D, chunks_per_tile, edge_split):
    mesh = plsc.VectorSubcoreMesh(core_axis_name="c", subcore_axis_name="s",
                                  num_cores=NC, num_subcores=NS)

    stage_chunks = 16
    stage_rows = ROWS_PER_TILE // stage_chunks

    def body(tables, eap, src_h, dst_h, init_h, out_h,
             src_v, dst_v, eap_v, rows_v, acc, sem):
        c = lax.axis_index("c")
        s = lax.axis_index("s")

        def init_chunk(i, carry):
            r = s * ROWS_PER_TILE + i * stage_rows
            pltpu.sync_copy(init_h.at[c].at[pl.ds(r, stage_rows)],
                            rows_v.at[pl.ds(0, stage_rows)])
            pltpu.sync_copy(rows_v.at[pl.ds(0, stage_rows)],
                            acc.at[pl.ds(r, stage_rows)])
            return carry
        lax.fori_loop(0, stage_chunks, init_chunk, 0)
        plsc.subcore_barrier()

        base0 = (c * NS + s) * chunks_per_tile if edge_split else s * chunks_per_tile

        def chunk(k, carry):
            b128 = (base0 + k) * K
            be = (base0 + k) * CE
            pltpu.sync_copy(src_h.at[pl.ds(b128, K)], src_v)
            pltpu.sync_copy(dst_h.at[pl.ds(b128, K)], dst_v)
            if edge_split:
                ecp = pltpu.async_copy(eap.at[pl.ds(be, CE)], eap_v, sem)
            else:
                ecp = pltpu.async_copy(eap.at[c, pl.ds(be, CE)], eap_v, sem)
            gathers = []
            for j in range(K):
                if edge_split:
                    g = pltpu.async_copy(tables.at[src_v.at[j]],
                                         rows_v.at[pl.ds(j * 128, 128)], sem)
                else:
                    g = pltpu.async_copy(tables.at[c].at[src_v.at[j]],
                                         rows_v.at[pl.ds(j * 128, 128)], sem)
                gathers.append(g)
            ecp.wait()
            for g in gathers:
                g.wait()

            def row(e, carry2):
                for dd in range(D // 16):
                    sl = pl.ds(dd * 16, 16)
                    rows_v[e, sl] = _tanh_vreg(rows_v[e, sl] + eap_v[e, sl])
                return carry2
            lax.fori_loop(0, CE, row, 0)

            for j in range(K):
                pltpu.sync_copy(rows_v.at[pl.ds(j * 128, 128)],
                                acc.at[dst_v.at[j]], add=True)
            return carry
        lax.fori_loop(0, chunks_per_tile, chunk, 0)

        plsc.subcore_barrier()

        def out_chunk(i, carry):
            r = s * ROWS_PER_TILE + i * stage_rows
            pltpu.sync_copy(acc.at[pl.ds(r, stage_rows)],
                            rows_v.at[pl.ds(0, stage_rows)])
            pltpu.sync_copy(rows_v.at[pl.ds(0, stage_rows)],
                            out_h.at[c].at[pl.ds(r, stage_rows)])
            return carry
        lax.fori_loop(0, stage_chunks, out_chunk, 0)

    return pl.kernel(
        body,
        out_type=jax.ShapeDtypeStruct((NC, NPAD, D), jnp.float32),
        mesh=mesh,
        compiler_params=pltpu.CompilerParams(use_tc_tiling_on_sc=False),
        scratch_types=[
            pltpu.VMEM((K, 128), jnp.int32),
            pltpu.VMEM((K, 128), jnp.int32),
            pltpu.VMEM((CE, D), jnp.float32),
            pltpu.VMEM((CE, D), jnp.float32),
            pltpu.VMEM_SHARED((NPAD, D), jnp.float32),
            pltpu.SemaphoreType.DMA,
        ],
    )


_sc_conv1 = _make_sc_conv(32, EP // (NS * CE), edge_split=False)
_sc_conv2 = _make_sc_conv(16, EP // (NC * NS * CE), edge_split=True)


def _head_body(emb_ref, W1_ref, b1_ref, W2_ref, b2_ref, W3_ref, b3_ref,
               Wo_ref, bo_ref, s1_ref, t1_ref, s2_ref, t2_ref, s3_ref, t3_ref,
               out_ref):
    o = jnp.maximum(emb_ref[...] @ W1_ref[...] + b1_ref[...], 0.0)
    o = o * s1_ref[...] + t1_ref[...]
    o = jnp.maximum(o @ W2_ref[...] + b2_ref[...], 0.0)
    o = o * s2_ref[...] + t2_ref[...]
    o = jnp.maximum(o @ W3_ref[...] + b3_ref[...], 0.0)
    o = o * s3_ref[...] + t3_ref[...]
    out_ref[...] = o @ Wo_ref[...] + bo_ref[...]


def _mlp_head(emb, W1, b1, W2, b2, W3, b3, Wo, bo, g1, be1, g2, be2, g3, be3):
    Gp = 1024
    embp = jnp.zeros((Gp, 16), jnp.float32).at[:G].set(emb)
    out = pl.pallas_call(
        _head_body,
        out_shape=jax.ShapeDtypeStruct((Gp, 1), jnp.float32),
    )(embp, W1, b1[None, :], W2, b2[None, :], W3, b3[None, :], Wo, bo[None, :],
      (g1 * _BN_SCALE)[None, :], be1[None, :],
      (g2 * _BN_SCALE)[None, :], be2[None, :],
      (g3 * _BN_SCALE)[None, :], be3[None, :])
    return out[:G, 0]


def kernel(x, edge_index, edge_attr, batch, W_neg1, b_neg1, W_root1, b_root1,
           W_neg2, b_neg2, W_root2, b_root2, Wg1, bg1, Wg2, bg2, Wg3, bg3,
           W1, b1, W2, b2, W3, b3, Wo, bo, g1, be1, g2, be2, g3, be3):
    src = edge_index[1]
    dst = edge_index[0]
    srcp = jnp.pad(src, (0, EP - E)).reshape(EP // 128, 128)
    dstp = jnp.pad(dst, (0, EP - E), constant_values=DUMMY_DST).reshape(EP // 128, 128)

    W1x = W_neg1[:39]
    W1e = W_neg1[39:]
    xp = x @ W1x + b_neg1
    xps = jnp.stack([xp[:, :32], xp[:, 32:]])
    eap1 = edge_attr @ W1e
    eap1p = jnp.pad(eap1, ((0, EP - E), (0, 0)))
    eaps1 = jnp.stack([eap1p[:, :32], eap1p[:, 32:]])
    root1 = jnp.tanh(x @ W_root1 + b_root1)
    root1p = jnp.pad(root1, ((0, NPAD - N), (0, 0)))
    init1 = jnp.stack([root1p[:, :32], root1p[:, 32:]])
    out1 = _sc_conv1(xps, eaps1, srcp, dstp, init1)
    h1 = jnp.concatenate([out1[0, :N], out1[1, :N]], axis=1)

    W2h = W_neg2[:64]
    W2e = W_neg2[64:]
    xp2 = h1 @ W2h + b_neg2
    eap2 = jnp.pad(edge_attr @ W2e, ((0, EP - E), (0, 0)))
    root2 = jnp.tanh(h1 @ W_root2 + b_root2)
    root2p = jnp.pad(root2, ((0, NPAD - N), (0, 0)))
    init2 = jnp.stack([root2p, jnp.zeros_like(root2p)])
    out2 = _sc_conv2(xp2, eap2, srcp, dstp, init2)
    h2 = out2[0, :N] + out2[1, :N]

    gate = jax.nn.relu(h2 @ Wg1 + bg1)
    gate = jax.nn.relu(gate @ Wg2 + bg2)
    gate = gate @ Wg3 + bg3
    gmax = jax.ops.segment_max(gate, batch, num_segments=G)
    gexp = jnp.exp(gate - gmax[batch])
    gden = jax.ops.segment_sum(gexp, batch, num_segments=G)
    att = gexp / (gden[batch] + 1e-16)
    emb = jax.ops.segment_sum(att * h2, batch, num_segments=G)

    o = _mlp_head(emb, W1, b1, W2, b2, W3, b3, Wo, bo,
                  g1, be1, g2, be2, g3, be3)
    return (o, att)


# speedup vs baseline: 3.1360x; 1.5896x over previous
import functools

import jax
import jax.numpy as jnp
from jax import lax
from jax.experimental import pallas as pl
from jax.experimental.pallas import tpu as pltpu
from jax.experimental.pallas import tpu_sc as plsc

N = 50000
E = 1600000
G = 1000

NC = 2
NS = 16
CE = 128
NPAD = 50176
EP = 1605632
ROWS_PER_TILE = NPAD // NS
DUMMY_DST = 50100

_BN_SCALE = 1.0 / (1.0 + 1e-5) ** 0.5


def _tanh_vreg(z):
    ez = jnp.exp(z + z)
    return 1.0 - 2.0 / (ez + 1.0)


def _make_sc_conv(D, chunks_per_tile, edge_split):
    mesh = plsc.VectorSubcoreMesh(core_axis_name="c", subcore_axis_name="s",
                                  num_cores=NC, num_subcores=NS)
    assert chunks_per_tile % 2 == 0
    stage_rows = 112
    stage_chunks = ROWS_PER_TILE // stage_rows

    def body(tables, eap, src_h, dst_h, init_h, out_h,
             src0, src1, dst0, dst1, eap0, eap1, rows0, rows1, acc,
             isem0, isem1, dsem0, dsem1):
        c = lax.axis_index("c")
        s = lax.axis_index("s")
        srcb = (src0, src1)
        dstb = (dst0, dst1)
        eapb = (eap0, eap1)
        rowsb = (rows0, rows1)
        isems = (isem0, isem1)
        dsems = (dsem0, dsem1)

        def tbl_at(idx_ref):
            return (tables if edge_split else tables.at[c]).at[idx_ref]

        def eap_at(chunk):
            sl = pl.ds(chunk * CE, CE)
            return eap.at[sl] if edge_split else eap.at[c].at[sl]

        def init_chunk(i, carry):
            r = s * ROWS_PER_TILE + i * stage_rows
            pltpu.sync_copy(init_h.at[c].at[pl.ds(r, stage_rows)],
                            rows0.at[pl.ds(0, stage_rows)])
            pltpu.sync_copy(rows0.at[pl.ds(0, stage_rows)],
                            acc.at[pl.ds(r, stage_rows)])
            return carry
        lax.fori_loop(0, stage_chunks, init_chunk, 0)
        plsc.subcore_barrier()

        base0 = (c * NS + s if edge_split else s) * chunks_per_tile

        def fetch_idx(chunk, b):
            row = base0 + chunk
            pltpu.async_copy(src_h.at[row], srcb[b], isems[b])
            pltpu.async_copy(dst_h.at[row], dstb[b], isems[b])

        def drain_isem(b):
            pltpu.make_async_copy(src_h.at[0], srcb[b], isems[b]).wait()
            pltpu.make_async_copy(dst_h.at[0], dstb[b], isems[b]).wait()

        def fetch_data(chunk, b):
            pltpu.async_copy(eap_at(chunk), eapb[b], dsems[b])
            pltpu.async_copy(tbl_at(srcb[b]), rowsb[b], dsems[b])

        def drain_dsem(b):
            pltpu.make_async_copy(eap_at(0), eapb[b], dsems[b]).wait()
            pltpu.make_async_copy(eap_at(0), rowsb[b], dsems[b]).wait()

        def compute_scatter(b):
            rows_v = rowsb[b]
            eap_v = eapb[b]

            def cbody(i, carry):
                for r in range(8):
                    e = i * 8 + r
                    for dd in range(D // 16):
                        sl = pl.ds(dd * 16, 16)
                        rows_v[e, sl] = _tanh_vreg(rows_v[e, sl] + eap_v[e, sl])
                return carry
            lax.fori_loop(0, CE // 8, cbody, 0)
            pltpu.sync_copy(rows_v, acc.at[dstb[b]], add=True)

        fetch_idx(0, 0)
        fetch_idx(1, 1)
        drain_isem(0)
        fetch_data(0, 0)

        def step(kk, carry):
            c0 = 2 * kk
            drain_isem(1)
            fetch_data(c0 + 1, 1)
            drain_dsem(0)
            compute_scatter(0)

            @pl.when(c0 + 2 < chunks_per_tile)
            def _():
                fetch_idx(c0 + 2, 0)

            @pl.when(c0 + 2 < chunks_per_tile)
            def _():
                drain_isem(0)
                fetch_data(c0 + 2, 0)
            drain_dsem(1)
            compute_scatter(1)

            @pl.when(c0 + 3 < chunks_per_tile)
            def _():
                fetch_idx(c0 + 3, 1)
            return carry
        lax.fori_loop(0, chunks_per_tile // 2, step, 0)

        plsc.subcore_barrier()

        def out_chunk(i, carry):
            r = s * ROWS_PER_TILE + i * stage_rows
            pltpu.sync_copy(acc.at[pl.ds(r, stage_rows)],
                            rows0.at[pl.ds(0, stage_rows)])
            pltpu.sync_copy(rows0.at[pl.ds(0, stage_rows)],
                            out_h.at[c].at[pl.ds(r, stage_rows)])
            return carry
        lax.fori_loop(0, stage_chunks, out_chunk, 0)

    return pl.kernel(
        body,
        out_type=jax.ShapeDtypeStruct((NC, NPAD, D), jnp.float32),
        mesh=mesh,
        compiler_params=pltpu.CompilerParams(use_tc_tiling_on_sc=False),
        scratch_types=[
            pltpu.VMEM((CE,), jnp.int32),
            pltpu.VMEM((CE,), jnp.int32),
            pltpu.VMEM((CE,), jnp.int32),
            pltpu.VMEM((CE,), jnp.int32),
            pltpu.VMEM((CE, D), jnp.float32),
            pltpu.VMEM((CE, D), jnp.float32),
            pltpu.VMEM((CE, D), jnp.float32),
            pltpu.VMEM((CE, D), jnp.float32),
            pltpu.VMEM_SHARED((NPAD, D), jnp.float32),
            pltpu.SemaphoreType.DMA,
            pltpu.SemaphoreType.DMA,
            pltpu.SemaphoreType.DMA,
            pltpu.SemaphoreType.DMA,
        ],
    )


_sc_conv1 = _make_sc_conv(32, EP // (NS * CE), edge_split=False)
_sc_conv2 = _make_sc_conv(16, EP // (NC * NS * CE), edge_split=True)


def _head_body(emb_ref, W1_ref, b1_ref, W2_ref, b2_ref, W3_ref, b3_ref,
               Wo_ref, bo_ref, s1_ref, t1_ref, s2_ref, t2_ref, s3_ref, t3_ref,
               out_ref):
    o = jnp.maximum(emb_ref[...] @ W1_ref[...] + b1_ref[...], 0.0)
    o = o * s1_ref[...] + t1_ref[...]
    o = jnp.maximum(o @ W2_ref[...] + b2_ref[...], 0.0)
    o = o * s2_ref[...] + t2_ref[...]
    o = jnp.maximum(o @ W3_ref[...] + b3_ref[...], 0.0)
    o = o * s3_ref[...] + t3_ref[...]
    out_ref[...] = o @ Wo_ref[...] + bo_ref[...]


def _mlp_head(emb, W1, b1, W2, b2, W3, b3, Wo, bo, g1, be1, g2, be2, g3, be3):
    Gp = 1024
    embp = jnp.zeros((Gp, 16), jnp.float32).at[:G].set(emb)
    out = pl.pallas_call(
        _head_body,
        out_shape=jax.ShapeDtypeStruct((Gp, 1), jnp.float32),
    )(embp, W1, b1[None, :], W2, b2[None, :], W3, b3[None, :], Wo, bo[None, :],
      (g1 * _BN_SCALE)[None, :], be1[None, :],
      (g2 * _BN_SCALE)[None, :], be2[None, :],
      (g3 * _BN_SCALE)[None, :], be3[None, :])
    return out[:G, 0]


def kernel(x, edge_index, edge_attr, batch, W_neg1, b_neg1, W_root1, b_root1,
           W_neg2, b_neg2, W_root2, b_root2, Wg1, bg1, Wg2, bg2, Wg3, bg3,
           W1, b1, W2, b2, W3, b3, Wo, bo, g1, be1, g2, be2, g3, be3):
    src = edge_index[1]
    dst = edge_index[0]
    srcp = jnp.pad(src, (0, EP - E)).reshape(EP // 128, 128)
    dstp = jnp.pad(dst, (0, EP - E), constant_values=DUMMY_DST).reshape(EP // 128, 128)

    W1x = W_neg1[:39]
    W1e = W_neg1[39:]
    xp = x @ W1x + b_neg1
    xps = jnp.stack([xp[:, :32], xp[:, 32:]])
    eap1 = edge_attr @ W1e
    eap1p = jnp.pad(eap1, ((0, EP - E), (0, 0)))
    eaps1 = jnp.stack([eap1p[:, :32], eap1p[:, 32:]])
    root1 = jnp.tanh(x @ W_root1 + b_root1)
    root1p = jnp.pad(root1, ((0, NPAD - N), (0, 0)))
    init1 = jnp.stack([root1p[:, :32], root1p[:, 32:]])
    out1 = _sc_conv1(xps, eaps1, srcp, dstp, init1)
    h1 = jnp.concatenate([out1[0, :N], out1[1, :N]], axis=1)

    W2h = W_neg2[:64]
    W2e = W_neg2[64:]
    xp2 = h1 @ W2h + b_neg2
    eap2 = jnp.pad(edge_attr @ W2e, ((0, EP - E), (0, 0)))
    root2 = jnp.tanh(h1 @ W_root2 + b_root2)
    root2p = jnp.pad(root2, ((0, NPAD - N), (0, 0)))
    init2 = jnp.stack([root2p, jnp.zeros_like(root2p)])
    out2 = _sc_conv2(xp2, eap2, srcp, dstp, init2)
    h2 = out2[0, :N] + out2[1, :N]

    gate = jax.nn.relu(h2 @ Wg1 + bg1)
    gate = jax.nn.relu(gate @ Wg2 + bg2)
    gate = gate @ Wg3 + bg3
    gmax = jax.ops.segment_max(gate, batch, num_segments=G)
    gexp = jnp.exp(gate - gmax[batch])
    gden = jax.ops.segment_sum(gexp, batch, num_segments=G)
    att = gexp / (gden[batch] + 1e-16)
    emb = jax.ops.segment_sum(att * h2, batch, num_segments=G)

    o = _mlp_head(emb, W1, b1, W2, b2, W3, b3, Wo, bo,
                  g1, be1, g2, be2, g3, be3)
    return (o, att)


# speedup vs baseline: 3.1421x; 1.0019x over previous
import functools

import jax
import jax.numpy as jnp
from jax import lax
from jax.experimental import pallas as pl
from jax.experimental.pallas import tpu as pltpu
from jax.experimental.pallas import tpu_sc as plsc

N = 50000
E = 1600000
G = 1000

NC = 2
NS = 16
CE = 128
NPAD = 50176
EP = 1605632
ROWS_PER_TILE = NPAD // NS
DUMMY_DST = 50100

_BN_SCALE = 1.0 / (1.0 + 1e-5) ** 0.5


def _tanh_vreg(z):
    ez = jnp.exp(z + z)
    return 1.0 - 2.0 / (ez + 1.0)


def _make_sc_conv(D, chunks_per_tile, edge_split):
    mesh = plsc.VectorSubcoreMesh(core_axis_name="c", subcore_axis_name="s",
                                  num_cores=NC, num_subcores=NS)
    assert chunks_per_tile % 2 == 0
    stage_rows = 112
    stage_chunks = ROWS_PER_TILE // stage_rows

    def body(tables, eap, src_h, dst_h, init_h, out_h,
             src0, src1, dst0, dst1, eap0, eap1, rows0, rows1, acc,
             isem0, isem1, dsem0, dsem1):
        c = lax.axis_index("c")
        s = lax.axis_index("s")
        srcb = (src0, src1)
        dstb = (dst0, dst1)
        eapb = (eap0, eap1)
        rowsb = (rows0, rows1)
        isems = (isem0, isem1)
        dsems = (dsem0, dsem1)

        def tbl_at(idx_ref):
            return (tables if edge_split else tables.at[c]).at[idx_ref]

        def eap_at(chunk):
            sl = pl.ds(chunk * CE, CE)
            return eap.at[sl] if edge_split else eap.at[c].at[sl]

        def init_chunk(i, carry):
            r = s * ROWS_PER_TILE + i * stage_rows
            pltpu.sync_copy(init_h.at[c].at[pl.ds(r, stage_rows)],
                            rows0.at[pl.ds(0, stage_rows)])
            pltpu.sync_copy(rows0.at[pl.ds(0, stage_rows)],
                            acc.at[pl.ds(r, stage_rows)])
            return carry
        lax.fori_loop(0, stage_chunks, init_chunk, 0)
        plsc.subcore_barrier()

        base0 = (c * NS + s if edge_split else s) * chunks_per_tile

        def fetch_idx(chunk, b):
            row = base0 + chunk
            pltpu.async_copy(src_h.at[row], srcb[b], isems[b])
            pltpu.async_copy(dst_h.at[row], dstb[b], isems[b])

        def drain_isem(b):
            pltpu.make_async_copy(src_h.at[0], srcb[b], isems[b]).wait()
            pltpu.make_async_copy(dst_h.at[0], dstb[b], isems[b]).wait()

        def fetch_data(chunk, b):
            pltpu.async_copy(eap_at(base0 + chunk), eapb[b], dsems[b])
            pltpu.async_copy(tbl_at(srcb[b]), rowsb[b], dsems[b])

        def drain_dsem(b):
            pltpu.make_async_copy(eap_at(0), eapb[b], dsems[b]).wait()
            pltpu.make_async_copy(eap_at(0), rowsb[b], dsems[b]).wait()

        def compute_scatter(b):
            rows_v = rowsb[b]
            eap_v = eapb[b]

            def cbody(i, carry):
                for r in range(8):
                    e = i * 8 + r
                    for dd in range(D // 16):
                        sl = pl.ds(dd * 16, 16)
                        rows_v[e, sl] = _tanh_vreg(rows_v[e, sl] + eap_v[e, sl])
                return carry
            lax.fori_loop(0, CE // 8, cbody, 0)
            pltpu.sync_copy(rows_v, acc.at[dstb[b]], add=True)

        fetch_idx(0, 0)
        fetch_idx(1, 1)
        drain_isem(0)
        fetch_data(0, 0)

        def step(kk, carry):
            c0 = 2 * kk
            drain_isem(1)
            fetch_data(c0 + 1, 1)
            drain_dsem(0)
            compute_scatter(0)

            @pl.when(c0 + 2 < chunks_per_tile)
            def _():
                fetch_idx(c0 + 2, 0)

            @pl.when(c0 + 2 < chunks_per_tile)
            def _():
                drain_isem(0)
                fetch_data(c0 + 2, 0)
            drain_dsem(1)
            compute_scatter(1)

            @pl.when(c0 + 3 < chunks_per_tile)
            def _():
                fetch_idx(c0 + 3, 1)
            return carry
        lax.fori_loop(0, chunks_per_tile // 2, step, 0)

        plsc.subcore_barrier()

        def out_chunk(i, carry):
            r = s * ROWS_PER_TILE + i * stage_rows
            pltpu.sync_copy(acc.at[pl.ds(r, stage_rows)],
                            rows0.at[pl.ds(0, stage_rows)])
            pltpu.sync_copy(rows0.at[pl.ds(0, stage_rows)],
                            out_h.at[c].at[pl.ds(r, stage_rows)])
            return carry
        lax.fori_loop(0, stage_chunks, out_chunk, 0)

    return pl.kernel(
        body,
        out_type=jax.ShapeDtypeStruct((NC, NPAD, D), jnp.float32),
        mesh=mesh,
        compiler_params=pltpu.CompilerParams(use_tc_tiling_on_sc=False),
        scratch_types=[
            pltpu.VMEM((CE,), jnp.int32),
            pltpu.VMEM((CE,), jnp.int32),
            pltpu.VMEM((CE,), jnp.int32),
            pltpu.VMEM((CE,), jnp.int32),
            pltpu.VMEM((CE, D), jnp.float32),
            pltpu.VMEM((CE, D), jnp.float32),
            pltpu.VMEM((CE, D), jnp.float32),
            pltpu.VMEM((CE, D), jnp.float32),
            pltpu.VMEM_SHARED((NPAD, D), jnp.float32),
            pltpu.SemaphoreType.DMA,
            pltpu.SemaphoreType.DMA,
            pltpu.SemaphoreType.DMA,
            pltpu.SemaphoreType.DMA,
        ],
    )


_sc_conv1 = _make_sc_conv(32, EP // (NS * CE), edge_split=False)
_sc_conv2 = _make_sc_conv(16, EP // (NC * NS * CE), edge_split=True)


def _head_body(emb_ref, W1_ref, b1_ref, W2_ref, b2_ref, W3_ref, b3_ref,
               Wo_ref, bo_ref, s1_ref, t1_ref, s2_ref, t2_ref, s3_ref, t3_ref,
               out_ref):
    o = jnp.maximum(emb_ref[...] @ W1_ref[...] + b1_ref[...], 0.0)
    o = o * s1_ref[...] + t1_ref[...]
    o = jnp.maximum(o @ W2_ref[...] + b2_ref[...], 0.0)
    o = o * s2_ref[...] + t2_ref[...]
    o = jnp.maximum(o @ W3_ref[...] + b3_ref[...], 0.0)
    o = o * s3_ref[...] + t3_ref[...]
    out_ref[...] = o @ Wo_ref[...] + bo_ref[...]


def _mlp_head(emb, W1, b1, W2, b2, W3, b3, Wo, bo, g1, be1, g2, be2, g3, be3):
    Gp = 1024
    embp = jnp.zeros((Gp, 16), jnp.float32).at[:G].set(emb)
    out = pl.pallas_call(
        _head_body,
        out_shape=jax.ShapeDtypeStruct((Gp, 1), jnp.float32),
    )(embp, W1, b1[None, :], W2, b2[None, :], W3, b3[None, :], Wo, bo[None, :],
      (g1 * _BN_SCALE)[None, :], be1[None, :],
      (g2 * _BN_SCALE)[None, :], be2[None, :],
      (g3 * _BN_SCALE)[None, :], be3[None, :])
    return out[:G, 0]


def kernel(x, edge_index, edge_attr, batch, W_neg1, b_neg1, W_root1, b_root1,
           W_neg2, b_neg2, W_root2, b_root2, Wg1, bg1, Wg2, bg2, Wg3, bg3,
           W1, b1, W2, b2, W3, b3, Wo, bo, g1, be1, g2, be2, g3, be3):
    src = edge_index[1]
    dst = edge_index[0]
    srcp = jnp.pad(src, (0, EP - E)).reshape(EP // 128, 128)
    dstp = jnp.pad(dst, (0, EP - E), constant_values=DUMMY_DST).reshape(EP // 128, 128)

    W1x = W_neg1[:39]
    W1e = W_neg1[39:]
    xp = x @ W1x + b_neg1
    xps = jnp.stack([xp[:, :32], xp[:, 32:]])
    eap1 = edge_attr @ W1e
    eap1p = jnp.pad(eap1, ((0, EP - E), (0, 0)))
    eaps1 = jnp.stack([eap1p[:, :32], eap1p[:, 32:]])
    root1 = jnp.tanh(x @ W_root1 + b_root1)
    root1p = jnp.pad(root1, ((0, NPAD - N), (0, 0)))
    init1 = jnp.stack([root1p[:, :32], root1p[:, 32:]])
    out1 = _sc_conv1(xps, eaps1, srcp, dstp, init1)
    h1 = jnp.concatenate([out1[0, :N], out1[1, :N]], axis=1)

    W2h = W_neg2[:64]
    W2e = W_neg2[64:]
    xp2 = h1 @ W2h + b_neg2
    eap2 = jnp.pad(edge_attr @ W2e, ((0, EP - E), (0, 0)))
    root2 = jnp.tanh(h1 @ W_root2 + b_root2)
    root2p = jnp.pad(root2, ((0, NPAD - N), (0, 0)))
    init2 = jnp.stack([root2p, jnp.zeros_like(root2p)])
    out2 = _sc_conv2(xp2, eap2, srcp, dstp, init2)
    h2 = out2[0, :N] + out2[1, :N]

    gate = jax.nn.relu(h2 @ Wg1 + bg1)
    gate = jax.nn.relu(gate @ Wg2 + bg2)
    gate = gate @ Wg3 + bg3
    gmax = jax.ops.segment_max(gate, batch, num_segments=G)
    gexp = jnp.exp(gate - gmax[batch])
    gden = jax.ops.segment_sum(gexp, batch, num_segments=G)
    att = gexp / (gden[batch] + 1e-16)
    emb = jax.ops.segment_sum(att * h2, batch, num_segments=G)

    o = _mlp_head(emb, W1, b1, W2, b2, W3, b3, Wo, bo,
                  g1, be1, g2, be2, g3, be3)
    return (o, att)


# speedup vs baseline: 3.6399x; 1.1584x over previous
import functools

import jax
import jax.numpy as jnp
from jax import lax
from jax.experimental import pallas as pl
from jax.experimental.pallas import tpu as pltpu
from jax.experimental.pallas import tpu_sc as plsc

N = 50000
E = 1600000
G = 1000

NC = 2
NS = 16
CE = 128
NPAD = 50176
EP = 1605632
ROWS_PER_TILE = NPAD // NS
DUMMY_DST = 50100

_BN_SCALE = 1.0 / (1.0 + 1e-5) ** 0.5


def _tanh_vreg(z):
    ez = jnp.exp(z + z)
    return 1.0 - 2.0 / (ez + 1.0)


def _make_sc_conv(D, chunks_per_tile, edge_split):
    mesh = plsc.VectorSubcoreMesh(core_axis_name="c", subcore_axis_name="s",
                                  num_cores=NC, num_subcores=NS)
    assert chunks_per_tile % 2 == 0
    stage_rows = 112
    stage_chunks = ROWS_PER_TILE // stage_rows

    def body(tables, eap, src_h, dst_h, init_h, out_h,
             src0, src1, dst0, dst1, eap0, eap1, rows0, rows1, acc,
             isem0, isem1, dsem0, dsem1):
        c = lax.axis_index("c")
        s = lax.axis_index("s")
        srcb = (src0, src1)
        dstb = (dst0, dst1)
        eapb = (eap0, eap1)
        rowsb = (rows0, rows1)
        isems = (isem0, isem1)
        dsems = (dsem0, dsem1)

        def tbl_at(idx_ref):
            return (tables if edge_split else tables.at[c]).at[idx_ref]

        def eap_at(chunk):
            sl = pl.ds(chunk * CE, CE)
            return eap.at[sl] if edge_split else eap.at[c].at[sl]

        def init_chunk(i, carry):
            r = s * ROWS_PER_TILE + i * stage_rows
            pltpu.sync_copy(init_h.at[c].at[pl.ds(r, stage_rows)],
                            rows0.at[pl.ds(0, stage_rows)])
            pltpu.sync_copy(rows0.at[pl.ds(0, stage_rows)],
                            acc.at[pl.ds(r, stage_rows)])
            return carry
        lax.fori_loop(0, stage_chunks, init_chunk, 0)
        plsc.subcore_barrier()

        base0 = (c * NS + s if edge_split else s) * chunks_per_tile

        def fetch_idx(chunk, b):
            row = base0 + chunk
            pltpu.async_copy(src_h.at[row], srcb[b], isems[b])
            pltpu.async_copy(dst_h.at[row], dstb[b], isems[b])

        def drain_isem(b):
            pltpu.make_async_copy(src_h.at[0], srcb[b], isems[b]).wait()
            pltpu.make_async_copy(dst_h.at[0], dstb[b], isems[b]).wait()

        def fetch_data(chunk, b):
            pltpu.async_copy(eap_at(base0 + chunk), eapb[b], dsems[b])
            pltpu.async_copy(tbl_at(srcb[b]), rowsb[b], dsems[b])

        def drain_dsem(b):
            pltpu.make_async_copy(eap_at(0), eapb[b], dsems[b]).wait()
            pltpu.make_async_copy(eap_at(0), rowsb[b], dsems[b]).wait()

        def compute_scatter(b):
            rows_v = rowsb[b]
            eap_v = eapb[b]

            def cbody(i, carry):
                for r in range(8):
                    e = i * 8 + r
                    for dd in range(D // 16):
                        sl = pl.ds(dd * 16, 16)
                        rows_v[e, sl] = _tanh_vreg(rows_v[e, sl] + eap_v[e, sl])
                return carry
            lax.fori_loop(0, CE // 8, cbody, 0)
            pltpu.sync_copy(rows_v, acc.at[dstb[b]], add=True)

        fetch_idx(0, 0)
        fetch_idx(1, 1)
        drain_isem(0)
        fetch_data(0, 0)

        def step(kk, carry):
            c0 = 2 * kk
            drain_isem(1)
            fetch_data(c0 + 1, 1)
            drain_dsem(0)
            compute_scatter(0)

            @pl.when(c0 + 2 < chunks_per_tile)
            def _():
                fetch_idx(c0 + 2, 0)

            @pl.when(c0 + 2 < chunks_per_tile)
            def _():
                drain_isem(0)
                fetch_data(c0 + 2, 0)
            drain_dsem(1)
            compute_scatter(1)

            @pl.when(c0 + 3 < chunks_per_tile)
            def _():
                fetch_idx(c0 + 3, 1)
            return carry
        lax.fori_loop(0, chunks_per_tile // 2, step, 0)

        plsc.subcore_barrier()

        def out_chunk(i, carry):
            r = s * ROWS_PER_TILE + i * stage_rows
            pltpu.sync_copy(acc.at[pl.ds(r, stage_rows)],
                            rows0.at[pl.ds(0, stage_rows)])
            pltpu.sync_copy(rows0.at[pl.ds(0, stage_rows)],
                            out_h.at[c].at[pl.ds(r, stage_rows)])
            return carry
        lax.fori_loop(0, stage_chunks, out_chunk, 0)

    return pl.kernel(
        body,
        out_type=jax.ShapeDtypeStruct((NC, NPAD, D), jnp.float32),
        mesh=mesh,
        compiler_params=pltpu.CompilerParams(use_tc_tiling_on_sc=False),
        scratch_types=[
            pltpu.VMEM((CE,), jnp.int32),
            pltpu.VMEM((CE,), jnp.int32),
            pltpu.VMEM((CE,), jnp.int32),
            pltpu.VMEM((CE,), jnp.int32),
            pltpu.VMEM((CE, D), jnp.float32),
            pltpu.VMEM((CE, D), jnp.float32),
            pltpu.VMEM((CE, D), jnp.float32),
            pltpu.VMEM((CE, D), jnp.float32),
            pltpu.VMEM_SHARED((NPAD, D), jnp.float32),
            pltpu.SemaphoreType.DMA,
            pltpu.SemaphoreType.DMA,
            pltpu.SemaphoreType.DMA,
            pltpu.SemaphoreType.DMA,
        ],
    )


_sc_conv1 = _make_sc_conv(32, EP // (NS * CE), edge_split=False)
_sc_conv2 = _make_sc_conv(16, EP // (NC * NS * CE), edge_split=True)

GP = 1024
RC = 112
NPT = NPAD // NS


def _readout_body(gsh_h, bat_h, h2_h, emb_h, att_h,
                  bat_v, g_v, gex_v, h2_v, valn_v, vald_v, att_v,
                  dbuf, nbuf, numer_sh, den_sh):
    c = lax.axis_index("c")
    s = lax.axis_index("s")
    zero16 = jnp.zeros((16,), jnp.float32)
    zidx = jnp.zeros((16,), jnp.int32)

    def z_row(r, carry):
        valn_v[r, pl.ds(0, 16)] = zero16
        return carry
    lax.fori_loop(0, GP // NS, z_row, 0)
    pltpu.sync_copy(valn_v.at[pl.ds(0, GP // NS)],
                    numer_sh.at[pl.ds(s * (GP // NS), GP // NS)])
    pltpu.sync_copy(valn_v.at[pl.ds(0, GP // NS)],
                    den_sh.at[pl.ds(s * (GP // NS), GP // NS)])
    plsc.subcore_barrier()

    def acc_chunk(i, carry):
        base = s * NPT + i * RC
        pltpu.sync_copy(gsh_h.at[pl.ds(base, RC)], g_v)
        pltpu.sync_copy(bat_h.at[pl.ds(base, RC)], bat_v)
        pltpu.sync_copy(h2_h.at[pl.ds(base, RC)], h2_v)

        def vexp(j, carry2):
            gex_v[pl.ds(j * 16, 16)] = jnp.exp(g_v[pl.ds(j * 16, 16)])
            return carry2
        lax.fori_loop(0, RC // 16, vexp, 0)

        def rowfill(j, carry2):
            ge = gex_v[pl.ds(j * 16, 16)]
            for r in range(16):
                e = j * 16 + r
                z = ge[r]
                valn_v[e, pl.ds(0, 16)] = h2_v[e, pl.ds(0, 16)] * z
                vald_v[e, pl.ds(0, 16)] = jnp.full((16,), z, jnp.float32)
            return carry2
        lax.fori_loop(0, RC // 16, rowfill, 0)

        pltpu.sync_copy(valn_v, numer_sh.at[bat_v], add=True)
        pltpu.sync_copy(vald_v, den_sh.at[bat_v], add=True)
        return carry
    lax.fori_loop(0, NPT // RC, acc_chunk, 0)
    plsc.subcore_barrier()

    pltpu.sync_copy(den_sh, dbuf)

    def att_chunk(i, carry):
        base = c * (NPAD // 2) + s * (NPAD // 2 // NS) + i * RC
        pltpu.sync_copy(gsh_h.at[pl.ds(base, RC)], g_v)
        pltpu.sync_copy(bat_h.at[pl.ds(base, RC)], bat_v)

        def vatt(j, carry2):
            sl = pl.ds(j * 16, 16)
            ge = jnp.exp(g_v[sl])
            den = plsc.load_gather(dbuf, [bat_v[sl], zidx])
            att_v[sl] = ge / (den + 1e-16)
            return carry2
        lax.fori_loop(0, RC // 16, vatt, 0)
        pltpu.sync_copy(att_v, att_h.at[pl.ds(base, RC)])
        return carry
    lax.fori_loop(0, NPAD // 2 // NS // RC, att_chunk, 0)

    @pl.when(c == 0)
    def _():
        r0 = s * (GP // NS)
        pltpu.sync_copy(numer_sh.at[pl.ds(r0, GP // NS)], nbuf)

        def erow(r, carry):
            dvec = dbuf[r0 + r, pl.ds(0, 16)]
            dv = jnp.full((16,), dvec[0], jnp.float32)
            nbuf[r, pl.ds(0, 16)] = nbuf[r, pl.ds(0, 16)] / (dv + 1e-16)
            return carry
        lax.fori_loop(0, GP // NS, erow, 0)
        pltpu.sync_copy(nbuf, emb_h.at[pl.ds(r0, GP // NS)])


_sc_readout = pl.kernel(
    _readout_body,
    out_type=(jax.ShapeDtypeStruct((GP, 16), jnp.float32),
              jax.ShapeDtypeStruct((NPAD,), jnp.float32)),
    mesh=plsc.VectorSubcoreMesh(core_axis_name="c", subcore_axis_name="s",
                                num_cores=NC, num_subcores=NS),
    compiler_params=pltpu.CompilerParams(use_tc_tiling_on_sc=False,
                                         needs_layout_passes=False),
    scratch_types=[
        pltpu.VMEM((RC,), jnp.int32),
        pltpu.VMEM((RC,), jnp.float32),
        pltpu.VMEM((RC,), jnp.float32),
        pltpu.VMEM((RC, 16), jnp.float32),
        pltpu.VMEM((RC, 16), jnp.float32),
        pltpu.VMEM((RC, 16), jnp.float32),
        pltpu.VMEM((RC,), jnp.float32),
        pltpu.VMEM((GP, 16), jnp.float32),
        pltpu.VMEM((GP // NS, 16), jnp.float32),
        pltpu.VMEM_SHARED((GP, 16), jnp.float32),
        pltpu.VMEM_SHARED((GP, 16), jnp.float32),
    ],
)


def _head_body(emb_ref, W1_ref, b1_ref, W2_ref, b2_ref, W3_ref, b3_ref,
               Wo_ref, bo_ref, s1_ref, t1_ref, s2_ref, t2_ref, s3_ref, t3_ref,
               out_ref):
    o = jnp.maximum(emb_ref[...] @ W1_ref[...] + b1_ref[...], 0.0)
    o = o * s1_ref[...] + t1_ref[...]
    o = jnp.maximum(o @ W2_ref[...] + b2_ref[...], 0.0)
    o = o * s2_ref[...] + t2_ref[...]
    o = jnp.maximum(o @ W3_ref[...] + b3_ref[...], 0.0)
    o = o * s3_ref[...] + t3_ref[...]
    out_ref[...] = o @ Wo_ref[...] + bo_ref[...]


def _mlp_head(emb, W1, b1, W2, b2, W3, b3, Wo, bo, g1, be1, g2, be2, g3, be3):
    Gp = 1024
    embp = jnp.zeros((Gp, 16), jnp.float32).at[:G].set(emb)
    out = pl.pallas_call(
        _head_body,
        out_shape=jax.ShapeDtypeStruct((Gp, 1), jnp.float32),
    )(embp, W1, b1[None, :], W2, b2[None, :], W3, b3[None, :], Wo, bo[None, :],
      (g1 * _BN_SCALE)[None, :], be1[None, :],
      (g2 * _BN_SCALE)[None, :], be2[None, :],
      (g3 * _BN_SCALE)[None, :], be3[None, :])
    return out[:G, 0]


def kernel(x, edge_index, edge_attr, batch, W_neg1, b_neg1, W_root1, b_root1,
           W_neg2, b_neg2, W_root2, b_root2, Wg1, bg1, Wg2, bg2, Wg3, bg3,
           W1, b1, W2, b2, W3, b3, Wo, bo, g1, be1, g2, be2, g3, be3):
    src = edge_index[1]
    dst = edge_index[0]
    srcp = jnp.pad(src, (0, EP - E)).reshape(EP // 128, 128)
    dstp = jnp.pad(dst, (0, EP - E), constant_values=DUMMY_DST).reshape(EP // 128, 128)

    W1x = W_neg1[:39]
    W1e = W_neg1[39:]
    xp = x @ W1x + b_neg1
    xps = jnp.stack([xp[:, :32], xp[:, 32:]])
    eap1 = edge_attr @ W1e
    eap1p = jnp.pad(eap1, ((0, EP - E), (0, 0)))
    eaps1 = jnp.stack([eap1p[:, :32], eap1p[:, 32:]])
    root1 = jnp.tanh(x @ W_root1 + b_root1)
    root1p = jnp.pad(root1, ((0, NPAD - N), (0, 0)))
    init1 = jnp.stack([root1p[:, :32], root1p[:, 32:]])
    out1 = _sc_conv1(xps, eaps1, srcp, dstp, init1)
    h1 = jnp.concatenate([out1[0, :N], out1[1, :N]], axis=1)

    W2h = W_neg2[:64]
    W2e = W_neg2[64:]
    xp2 = h1 @ W2h + b_neg2
    eap2 = jnp.pad(edge_attr @ W2e, ((0, EP - E), (0, 0)))
    root2 = jnp.tanh(h1 @ W_root2 + b_root2)
    root2p = jnp.pad(root2, ((0, NPAD - N), (0, 0)))
    init2 = jnp.stack([root2p, jnp.zeros_like(root2p)])
    out2 = _sc_conv2(xp2, eap2, srcp, dstp, init2)
    h2 = out2[0, :N] + out2[1, :N]

    gate = jax.nn.relu(h2 @ Wg1 + bg1)
    gate = jax.nn.relu(gate @ Wg2 + bg2)
    gate = (gate @ Wg3 + bg3)[:, 0]
    gsh = jnp.pad(gate - jnp.max(gate), (0, NPAD - N), constant_values=-30.0)
    batp = jnp.pad(batch, (0, NPAD - N), constant_values=G)
    h2p = jnp.pad(h2, ((0, NPAD - N), (0, 0)))
    embf, attf = _sc_readout(gsh, batp, h2p)
    emb = embf[:G]
    att = attf[:N, None]

    o = _mlp_head(emb, W1, b1, W2, b2, W3, b3, Wo, bo,
                  g1, be1, g2, be2, g3, be3)
    return (o, att)
